# baseline (speedup 1.0000x reference)
import functools
import math

import jax
import jax.numpy as jnp
from jax import lax
from jax.experimental import pallas as pl
from jax.experimental.pallas import tpu as pltpu
from jax.experimental.pallas import tpu_sc as plsc

N = 10000
NP = 10112
E = 320000
D = 128
DEPTH = 4
BN_EPS = 1e-5

NSC = 16
CH = 128
NCH = 158
EPT = CH * NCH
EP = NSC * EPT
ROWS_T = NP // NSC

@functools.cache
def _mesh():
    return plsc.VectorSubcoreMesh(core_axis_name="c", subcore_axis_name="s",
                                  num_cores=2, num_subcores=NSC)


def _sc_dual_propagate(tab_a, tab_b, idx2, zblk):
    @functools.partial(
        pl.kernel,
        out_type=(jax.ShapeDtypeStruct((NP, D), jnp.float32),
                  jax.ShapeDtypeStruct((NP, D), jnp.float32)),
        mesh=_mesh(),
        scratch_types=[
            pltpu.VMEM_SHARED((NP, D), jnp.float32),
            pltpu.VMEM((3, 2, CH), jnp.int32),
            pltpu.VMEM((2, CH, D), jnp.float32),
            pltpu.SemaphoreType.DMA,
            pltpu.SemaphoreType.DMA,
            pltpu.SemaphoreType.DMA,
        ],
    )
    def k(ta, tb, ix, zb, out_a, out_b, acc, islot, rows, isem, gsem, ssem):
        c = lax.axis_index("c")
        s = lax.axis_index("s")
        pltpu.sync_copy(zb, acc.at[pl.ds(s * ROWS_T, ROWS_T)])
        pltpu.sync_copy(ix.at[s, 0], islot.at[0])
        pltpu.sync_copy(ix.at[s, 1], islot.at[1])
        plsc.subcore_barrier()

        def run(table, out):
            pltpu.async_copy(table.at[islot.at[0, 0]], rows.at[0], gsem)

            @pl.loop(0, NCH)
            def _(g):
                b = lax.rem(g, 2)
                m = lax.rem(g, 3)
                pltpu.make_async_copy(table.at[pl.ds(0, CH)], rows.at[b],
                                      gsem).wait()
                pltpu.async_copy(rows.at[b], acc.at[islot.at[m, 1]], ssem,
                                 add=True)

                @pl.when(g >= 1)
                def _():
                    pltpu.make_async_copy(table.at[pl.ds(0, CH)], rows.at[b],
                                          ssem).wait()

                @pl.when(g + 1 < NCH)
                def _():
                    @pl.when(g >= 1)
                    def _():
                        pltpu.make_async_copy(ix.at[s, 0], islot.at[0],
                                              isem).wait()
                    pltpu.async_copy(
                        table.at[islot.at[lax.rem(g + 1, 3), 0]],
                        rows.at[1 - b], gsem)

                @pl.when(g + 2 < NCH)
                def _():
                    pltpu.async_copy(ix.at[s, g + 2],
                                     islot.at[lax.rem(g + 2, 3)], isem)

            pltpu.make_async_copy(table.at[pl.ds(0, CH)], rows.at[0],
                                  ssem).wait()
            plsc.subcore_barrier()
            pltpu.sync_copy(acc.at[pl.ds(s * ROWS_T, ROWS_T)],
                            out.at[pl.ds(s * ROWS_T, ROWS_T)])

        @pl.when(c == 0)
        def _():
            run(ta, out_a)

        @pl.when(c == 1)
        def _():
            run(tb, out_b)

    return k(tab_a, tab_b, idx2, zblk)


def _sc_split_propagate(tab, idx2, zblk):
    NH = NCH // 2

    @functools.partial(
        pl.kernel,
        out_type=(jax.ShapeDtypeStruct((NP, D), jnp.float32),
                  jax.ShapeDtypeStruct((NP, D), jnp.float32)),
        mesh=_mesh(),
        scratch_types=[
            pltpu.VMEM_SHARED((NP, D), jnp.float32),
            pltpu.VMEM((3, 2, CH), jnp.int32),
            pltpu.VMEM((2, CH, D), jnp.float32),
            pltpu.SemaphoreType.DMA,
            pltpu.SemaphoreType.DMA,
            pltpu.SemaphoreType.DMA,
        ],
    )
    def k(t, ix, zb, out_a, out_b, acc, islot, rows, isem, gsem, ssem):
        c = lax.axis_index("c")
        s = lax.axis_index("s")
        base = c * NH
        pltpu.sync_copy(zb, acc.at[pl.ds(s * ROWS_T, ROWS_T)])
        pltpu.sync_copy(ix.at[s, base], islot.at[0])
        pltpu.sync_copy(ix.at[s, base + 1], islot.at[1])
        plsc.subcore_barrier()
        pltpu.async_copy(t.at[islot.at[0, 0]], rows.at[0], gsem)

        @pl.loop(0, NH)
        def _(g):
            b = lax.rem(g, 2)
            m = lax.rem(g, 3)
            pltpu.make_async_copy(t.at[pl.ds(0, CH)], rows.at[b],
                                  gsem).wait()
            pltpu.async_copy(rows.at[b], acc.at[islot.at[m, 1]], ssem,
                             add=True)

            @pl.when(g >= 1)
            def _():
                pltpu.make_async_copy(t.at[pl.ds(0, CH)], rows.at[b],
                                      ssem).wait()

            @pl.when(g + 1 < NH)
            def _():
                @pl.when(g >= 1)
                def _():
                    pltpu.make_async_copy(ix.at[s, 0], islot.at[0],
                                          isem).wait()
                pltpu.async_copy(t.at[islot.at[lax.rem(g + 1, 3), 0]],
                                 rows.at[1 - b], gsem)

            @pl.when(g + 2 < NH)
            def _():
                pltpu.async_copy(ix.at[s, base + g + 2],
                                 islot.at[lax.rem(g + 2, 3)], isem)

        pltpu.make_async_copy(t.at[pl.ds(0, CH)], rows.at[0], ssem).wait()
        plsc.subcore_barrier()
        dst = s * ROWS_T

        @pl.when(c == 0)
        def _():
            pltpu.sync_copy(acc.at[pl.ds(dst, ROWS_T)],
                            out_a.at[pl.ds(dst, ROWS_T)])

        @pl.when(c == 1)
        def _():
            pltpu.sync_copy(acc.at[pl.ds(dst, ROWS_T)],
                            out_b.at[pl.ds(dst, ROWS_T)])

    return k(tab, idx2, zblk)


def _sc_degree(dstp, srcp, ones_h, z_h):
    @functools.partial(
        pl.kernel,
        out_type=jax.ShapeDtypeStruct((2, NP, D), jnp.float32),
        mesh=_mesh(),
        scratch_types=[
            pltpu.VMEM_SHARED((NP, D), jnp.float32),
            pltpu.VMEM((CH, D), jnp.float32),
            pltpu.VMEM((NCH, CH), jnp.int32),
            pltpu.SemaphoreType.DMA,
        ],
    )
    def k(di, si, oh, zh, out, acc, ones, sbuf, ssem):
        c = lax.axis_index("c")
        s = lax.axis_index("s")
        pltpu.sync_copy(oh, ones)
        pltpu.sync_copy(zh, acc.at[pl.ds(s * ROWS_T, ROWS_T)])

        def run(idx, cslot):
            pltpu.sync_copy(idx.at[s], sbuf)
            plsc.subcore_barrier()

            @pl.loop(0, NCH)
            def _(g):
                pltpu.async_copy(ones, acc.at[sbuf.at[g]], ssem, add=True)

                @pl.when(g >= 1)
                def _():
                    pltpu.make_async_copy(oh, ones, ssem).wait()

            pltpu.make_async_copy(oh, ones, ssem).wait()
            plsc.subcore_barrier()
            pltpu.sync_copy(acc.at[pl.ds(s * ROWS_T, ROWS_T)],
                            out.at[cslot, pl.ds(s * ROWS_T, ROWS_T)])

        @pl.when(c == 0)
        def _():
            run(di, 0)

        @pl.when(c == 1)
        def _():
            run(si, 1)

    return k(dstp, srcp, ones_h, z_h)


_RM = 2528
_RE = 2528
_RH = 2000


def _rb(r, w=D):
    return pl.BlockSpec((r, w), lambda i: (i, 0))


def _full(shape):
    return pl.BlockSpec(shape, lambda i: tuple(0 for _ in shape))


def _tc_degree(counts):
    def body(c_ref, dinv_ref, cnt_ref, cntc_ref, wsum_ref):
        cb = c_ref[...]
        deg = cb[0, :, 0:1] + 1.0
        cnt = cb[1, :, 0:1]
        dinv_ref[...] = lax.rsqrt(deg)
        cnt_ref[...] = cnt
        cntc_ref[...] = jnp.maximum(cnt, 1.0)
        wsum_ref[...] = cnt + deg - 1.0

    o = jax.ShapeDtypeStruct((NP, 1), jnp.float32)
    return pl.pallas_call(
        body, grid=(NP // _RE,),
        in_specs=[pl.BlockSpec((2, _RE, D), lambda i: (0, i, 0))],
        out_specs=[_rb(_RE, 1)] * 4,
        out_shape=[o] * 4,
    )(counts)


def _tc_pre(x, w, b, dinv):
    def body(x_ref, w_ref, b_ref, dinv_ref, h_ref, t_ref):
        h = jax.nn.relu(
            jnp.dot(x_ref[...], w_ref[...],
                    preferred_element_type=jnp.float32) + b_ref[...])
        h_ref[...] = h
        t_ref[...] = h * dinv_ref[...]

    o = jax.ShapeDtypeStruct((NP, D), jnp.float32)
    return pl.pallas_call(
        body, grid=(NP // _RM,),
        in_specs=[_rb(_RM), _full((D, D)), _full((1, D)), _rb(_RM, 1)],
        out_specs=[_rb(_RM)] * 2,
        out_shape=[o] * 2,
    )(x, w, b, dinv)


def _tc_layer1(p0, p1, t, dinv, wc, wg, bc, bg):
    def body(p0_ref, p1_ref, t_ref, dinv_ref, wc_ref, wg_ref, bc_ref,
             bg_ref, x_ref, h2_ref, h2sq_ref):
        m = (p0_ref[...] + p1_ref[...] + t_ref[...]) * dinv_ref[...]
        x_ref[...] = jax.nn.relu(
            jnp.dot(m, wc_ref[...], preferred_element_type=jnp.float32,
                    precision=lax.Precision.HIGHEST) + bc_ref[...])
        h2 = jax.nn.relu(
            jnp.dot(m, wg_ref[...], preferred_element_type=jnp.float32,
                    precision=lax.Precision.HIGHEST) + bg_ref[...])
        h2_ref[...] = h2
        h2sq_ref[...] = h2 * h2

    o = jax.ShapeDtypeStruct((NP, D), jnp.float32)
    return pl.pallas_call(
        body, grid=(NP // _RM,),
        in_specs=[_rb(_RM), _rb(_RM), _rb(_RM), _rb(_RM, 1),
                  _full((D, D)), _full((D, D)), _full((1, D)),
                  _full((1, D))],
        out_specs=[_rb(_RM)] * 3,
        out_shape=[o] * 3,
    )(p0, p1, t, dinv, wc, wg, bc, bg)


def _tc_epi2(u, w, h2, h2sq, cnt, cntc, h, x_, dinv):
    def body(u_ref, w_ref, h2_ref, h2sq_ref, cnt_ref, cntc_ref, h_ref,
             x_ref, dinv_ref, o_ref, t_ref):
        h2 = h2_ref[...]
        s = cnt_ref[...] * h2sq_ref[...] - 2.0 * h2 * u_ref[...] + w_ref[...]
        tau = jnp.tanh(s / cntc_ref[...])
        hb = h_ref[...]
        hn = hb + tau * (x_ref[...] - hb)
        o_ref[...] = hn
        t_ref[...] = hn * dinv_ref[...]

    o = jax.ShapeDtypeStruct((NP, D), jnp.float32)
    return pl.pallas_call(
        body, grid=(NP // _RE,),
        in_specs=[_rb(_RE), _rb(_RE), _rb(_RE), _rb(_RE), _rb(_RE, 1),
                  _rb(_RE, 1), _rb(_RE), _rb(_RE), _rb(_RE, 1)],
        out_specs=[_rb(_RE)] * 2,
        out_shape=[o] * 2,
    )(u, w, h2, h2sq, cnt, cntc, h, x_, dinv)


def _tc_head(h, p):
    bn_sc = 1.0 / math.sqrt(1.0 + BN_EPS)

    def body(h_ref, w1, b1, w2, b2, w3, b3, w4, b4, g_ref, be_ref, o_ref):
        g = g_ref[...] * bn_sc
        be = be_ref[...]
        o = jnp.dot(h_ref[...], w1[...],
                    preferred_element_type=jnp.float32) + b1[...]
        o = jax.nn.relu(g * o + be)
        o = jnp.dot(o, w2[...], preferred_element_type=jnp.float32) + b2[...]
        o = jax.nn.relu(g * o + be)
        o = jnp.dot(o, w3[...], preferred_element_type=jnp.float32) + b3[...]
        o = jax.nn.relu(g * o + be)
        o_ref[...] = jnp.dot(o, w4[...],
                             preferred_element_type=jnp.float32) + b4[...]

    return pl.pallas_call(
        body, grid=(N // _RH,),
        in_specs=[_rb(_RH), _full((D, D)), _full((1, D)), _full((D, D)),
                  _full((1, D)), _full((D, D)), _full((1, D)),
                  _full((D, 1)), _full((1, 1)), _full((1, D)), _full((1, D))],
        out_specs=_rb(_RH, 1),
        out_shape=jax.ShapeDtypeStruct((N, 1), jnp.float32),
    )(h, p['lin1_w'], p['lin1_b'].reshape(1, D),
      p['lin2_w'], p['lin2_b'].reshape(1, D),
      p['lin3a_w'], p['lin3a_b'].reshape(1, D),
      p['lin4a_w'], p['lin4a_b'].reshape(1, 1),
      p['bn_g'].reshape(1, D), p['bn_b'].reshape(1, D))


def _tc_dirichlet(h, r0, r1, wsum):
    def body(h_ref, r0_ref, r1_ref, w_ref, o_ref):
        i = pl.program_id(0)
        hb = h_ref[...]
        part = (jnp.sum(w_ref[...] * jnp.sum(hb * hb, axis=1, keepdims=True))
                - 2.0 * jnp.sum(hb * (r0_ref[...] + r1_ref[...])))

        @pl.when(i == 0)
        def _():
            o_ref[...] = jnp.zeros((1, 1), jnp.float32)

        o_ref[...] += part

        @pl.when(i == pl.num_programs(0) - 1)
        def _():
            o_ref[...] = o_ref[...] * (0.5 / float(E))

    return pl.pallas_call(
        body, grid=(N // _RH,),
        in_specs=[_rb(_RH), _rb(_RH), _rb(_RH), _rb(_RH, 1)],
        out_specs=pl.BlockSpec((1, 1), lambda i: (0, 0)),
        out_shape=jax.ShapeDtypeStruct((1, 1), jnp.float32),
    )(h, r0, r1, wsum)


def kernel(x, edge_index, params):
    src = edge_index[0]
    dst = edge_index[1]
    pad = jnp.full((EP - E,), N, jnp.int32)
    srcp = jnp.concatenate([src, pad]).reshape(NSC, NCH, CH)
    dstp = jnp.concatenate([dst, pad]).reshape(NSC, NCH, CH)
    ifwd = jnp.stack([srcp, dstp], axis=2)
    irev = jnp.stack([dstp, srcp], axis=2)
    xp = jnp.pad(x, ((0, NP - N), (0, 0)))
    zblk = jnp.zeros((ROWS_T, D), jnp.float32)

    counts = _sc_degree(dstp, srcp, jnp.ones((CH, D), jnp.float32), zblk)
    dinv, cnt, cntc, wsum = _tc_degree(counts)

    h, t = _tc_pre(xp, params['pre_w'], params['pre_b'].reshape(1, D), dinv)
    for i in range(DEPTH):
        p0, p1 = _sc_split_propagate(t, ifwd, zblk)
        x_, h2, h2sq = _tc_layer1(p0, p1, t, dinv,
                                  params['conv_w'][i], params['gg_w'][i],
                                  params['conv_b'][i].reshape(1, D),
                                  params['gg_b'][i].reshape(1, D))
        u, w = _sc_dual_propagate(h2, h2sq, irev, zblk)
        h, t = _tc_epi2(u, w, h2, h2sq, cnt, cntc, h, x_, dinv)

    r0, r1 = _sc_split_propagate(h, irev, zblk)
    out = _tc_head(h, params)
    de = _tc_dirichlet(h, r0, r1, wsum)
    return (out, out, de[0, 0])


# speedup vs baseline: 1.0809x; 1.0809x over previous
import functools
import math

import jax
import jax.numpy as jnp
from jax import lax
from jax.experimental import pallas as pl
from jax.experimental.pallas import tpu as pltpu
from jax.experimental.pallas import tpu_sc as plsc

N = 10000
NP = 10112
E = 320000
D = 128
DEPTH = 4
BN_EPS = 1e-5

NSC = 16
CH = 64
NCH = 316
EPT = CH * NCH
EP = NSC * EPT
ROWS_T = NP // NSC
NRB = 4
NIS = 5

@functools.cache
def _mesh():
    return plsc.VectorSubcoreMesh(core_axis_name="c", subcore_axis_name="s",
                                  num_cores=2, num_subcores=NSC)


def _pipe(table, ix, s, base, nch, acc, islot, rows, isem, gsem, ssem):
    pltpu.sync_copy(ix.at[s, base], islot.at[0])
    pltpu.sync_copy(ix.at[s, base + 1], islot.at[1])
    pltpu.sync_copy(ix.at[s, base + 2], islot.at[2])
    pltpu.async_copy(table.at[islot.at[0, 0]], rows.at[0], gsem)
    pltpu.async_copy(table.at[islot.at[1, 0]], rows.at[1], gsem)

    @pl.loop(0, nch)
    def _(g):
        b = lax.rem(g, NRB)
        m = lax.rem(g, NIS)
        pltpu.make_async_copy(table.at[pl.ds(0, CH)], rows.at[b],
                              gsem).wait()
        pltpu.async_copy(rows.at[b], acc.at[islot.at[m, 1]], ssem,
                         add=True)

        @pl.when(g >= 2)
        def _():
            pltpu.make_async_copy(table.at[pl.ds(0, CH)], rows.at[b],
                                  ssem).wait()

        @pl.when(g + 2 < nch)
        def _():
            @pl.when(g >= 1)
            def _():
                pltpu.make_async_copy(ix.at[s, 0], islot.at[0],
                                      isem).wait()
            pltpu.async_copy(table.at[islot.at[lax.rem(g + 2, NIS), 0]],
                             rows.at[lax.rem(g + 2, NRB)], gsem)

        @pl.when(g + 3 < nch)
        def _():
            pltpu.async_copy(ix.at[s, base + g + 3],
                             islot.at[lax.rem(g + 3, NIS)], isem)

    pltpu.make_async_copy(table.at[pl.ds(0, CH)], rows.at[0], ssem).wait()
    pltpu.make_async_copy(table.at[pl.ds(0, CH)], rows.at[0], ssem).wait()


def _sc_dual_propagate(tab_a, tab_b, idx2, zblk):
    @functools.partial(
        pl.kernel,
        out_type=(jax.ShapeDtypeStruct((NP, D), jnp.float32),
                  jax.ShapeDtypeStruct((NP, D), jnp.float32)),
        mesh=_mesh(),
        scratch_types=[
            pltpu.VMEM_SHARED((NP, D), jnp.float32),
            pltpu.VMEM((NIS, 2, CH), jnp.int32),
            pltpu.VMEM((NRB, CH, D), jnp.float32),
            pltpu.SemaphoreType.DMA,
            pltpu.SemaphoreType.DMA,
            pltpu.SemaphoreType.DMA,
        ],
    )
    def k(ta, tb, ix, zb, out_a, out_b, acc, islot, rows, isem, gsem, ssem):
        c = lax.axis_index("c")
        s = lax.axis_index("s")
        pltpu.sync_copy(zb, acc.at[pl.ds(s * ROWS_T, ROWS_T)])
        plsc.subcore_barrier()

        def run(table, out):
            _pipe(table, ix, s, 0, NCH, acc, islot, rows, isem, gsem, ssem)
            plsc.subcore_barrier()
            pltpu.sync_copy(acc.at[pl.ds(s * ROWS_T, ROWS_T)],
                            out.at[pl.ds(s * ROWS_T, ROWS_T)])

        @pl.when(c == 0)
        def _():
            run(ta, out_a)

        @pl.when(c == 1)
        def _():
            run(tb, out_b)

    return k(tab_a, tab_b, idx2, zblk)


def _sc_split_propagate(tab, idx2, zblk):
    NH = NCH // 2

    @functools.partial(
        pl.kernel,
        out_type=(jax.ShapeDtypeStruct((NP, D), jnp.float32),
                  jax.ShapeDtypeStruct((NP, D), jnp.float32)),
        mesh=_mesh(),
        scratch_types=[
            pltpu.VMEM_SHARED((NP, D), jnp.float32),
            pltpu.VMEM((NIS, 2, CH), jnp.int32),
            pltpu.VMEM((NRB, CH, D), jnp.float32),
            pltpu.SemaphoreType.DMA,
            pltpu.SemaphoreType.DMA,
            pltpu.SemaphoreType.DMA,
        ],
    )
    def k(t, ix, zb, out_a, out_b, acc, islot, rows, isem, gsem, ssem):
        c = lax.axis_index("c")
        s = lax.axis_index("s")
        pltpu.sync_copy(zb, acc.at[pl.ds(s * ROWS_T, ROWS_T)])
        plsc.subcore_barrier()
        _pipe(t, ix, s, c * NH, NH, acc, islot, rows, isem, gsem, ssem)
        plsc.subcore_barrier()
        dst = s * ROWS_T

        @pl.when(c == 0)
        def _():
            pltpu.sync_copy(acc.at[pl.ds(dst, ROWS_T)],
                            out_a.at[pl.ds(dst, ROWS_T)])

        @pl.when(c == 1)
        def _():
            pltpu.sync_copy(acc.at[pl.ds(dst, ROWS_T)],
                            out_b.at[pl.ds(dst, ROWS_T)])

    return k(tab, idx2, zblk)


def _sc_degree(dstp, srcp, ones_h, z_h):
    @functools.partial(
        pl.kernel,
        out_type=jax.ShapeDtypeStruct((2, NP, D), jnp.float32),
        mesh=_mesh(),
        scratch_types=[
            pltpu.VMEM_SHARED((NP, D), jnp.float32),
            pltpu.VMEM((CH, D), jnp.float32),
            pltpu.VMEM((NCH, CH), jnp.int32),
            pltpu.SemaphoreType.DMA,
        ],
    )
    def k(di, si, oh, zh, out, acc, ones, sbuf, ssem):
        c = lax.axis_index("c")
        s = lax.axis_index("s")
        pltpu.sync_copy(oh, ones)
        pltpu.sync_copy(zh, acc.at[pl.ds(s * ROWS_T, ROWS_T)])

        def run(idx, cslot):
            pltpu.sync_copy(idx.at[s], sbuf)
            plsc.subcore_barrier()

            @pl.loop(0, NCH)
            def _(g):
                pltpu.async_copy(ones, acc.at[sbuf.at[g]], ssem, add=True)

                @pl.when(g >= 1)
                def _():
                    pltpu.make_async_copy(oh, ones, ssem).wait()

            pltpu.make_async_copy(oh, ones, ssem).wait()
            plsc.subcore_barrier()
            pltpu.sync_copy(acc.at[pl.ds(s * ROWS_T, ROWS_T)],
                            out.at[cslot, pl.ds(s * ROWS_T, ROWS_T)])

        @pl.when(c == 0)
        def _():
            run(di, 0)

        @pl.when(c == 1)
        def _():
            run(si, 1)

    return k(dstp, srcp, ones_h, z_h)


_RM = 2528
_RE = 2528
_RH = 2000


def _rb(r, w=D):
    return pl.BlockSpec((r, w), lambda i: (i, 0))


def _full(shape):
    return pl.BlockSpec(shape, lambda i: tuple(0 for _ in shape))


def _tc_degree(counts):
    def body(c_ref, dinv_ref, cnt_ref, cntc_ref, wsum_ref):
        cb = c_ref[...]
        deg = cb[0, :, 0:1] + 1.0
        cnt = cb[1, :, 0:1]
        dinv_ref[...] = lax.rsqrt(deg)
        cnt_ref[...] = cnt
        cntc_ref[...] = jnp.maximum(cnt, 1.0)
        wsum_ref[...] = cnt + deg - 1.0

    o = jax.ShapeDtypeStruct((NP, 1), jnp.float32)
    return pl.pallas_call(
        body, grid=(NP // _RE,),
        in_specs=[pl.BlockSpec((2, _RE, D), lambda i: (0, i, 0))],
        out_specs=[_rb(_RE, 1)] * 4,
        out_shape=[o] * 4,
    )(counts)


def _tc_pre(x, w, b, dinv):
    def body(x_ref, w_ref, b_ref, dinv_ref, h_ref, t_ref):
        h = jax.nn.relu(
            jnp.dot(x_ref[...], w_ref[...],
                    preferred_element_type=jnp.float32) + b_ref[...])
        h_ref[...] = h
        t_ref[...] = h * dinv_ref[...]

    o = jax.ShapeDtypeStruct((NP, D), jnp.float32)
    return pl.pallas_call(
        body, grid=(NP // _RM,),
        in_specs=[_rb(_RM), _full((D, D)), _full((1, D)), _rb(_RM, 1)],
        out_specs=[_rb(_RM)] * 2,
        out_shape=[o] * 2,
    )(x, w, b, dinv)


def _tc_layer1(p0, p1, t, dinv, wc, wg, bc, bg):
    def body(p0_ref, p1_ref, t_ref, dinv_ref, wc_ref, wg_ref, bc_ref,
             bg_ref, x_ref, h2_ref, h2sq_ref):
        m = (p0_ref[...] + p1_ref[...] + t_ref[...]) * dinv_ref[...]
        x_ref[...] = jax.nn.relu(
            jnp.dot(m, wc_ref[...], preferred_element_type=jnp.float32,
                    precision=lax.Precision.HIGHEST) + bc_ref[...])
        h2 = jax.nn.relu(
            jnp.dot(m, wg_ref[...], preferred_element_type=jnp.float32,
                    precision=lax.Precision.HIGHEST) + bg_ref[...])
        h2_ref[...] = h2
        h2sq_ref[...] = h2 * h2

    o = jax.ShapeDtypeStruct((NP, D), jnp.float32)
    return pl.pallas_call(
        body, grid=(NP // _RM,),
        in_specs=[_rb(_RM), _rb(_RM), _rb(_RM), _rb(_RM, 1),
                  _full((D, D)), _full((D, D)), _full((1, D)),
                  _full((1, D))],
        out_specs=[_rb(_RM)] * 3,
        out_shape=[o] * 3,
    )(p0, p1, t, dinv, wc, wg, bc, bg)


def _tc_epi2(u, w, h2, h2sq, cnt, cntc, h, x_, dinv):
    def body(u_ref, w_ref, h2_ref, h2sq_ref, cnt_ref, cntc_ref, h_ref,
             x_ref, dinv_ref, o_ref, t_ref):
        h2 = h2_ref[...]
        s = cnt_ref[...] * h2sq_ref[...] - 2.0 * h2 * u_ref[...] + w_ref[...]
        tau = jnp.tanh(s / cntc_ref[...])
        hb = h_ref[...]
        hn = hb + tau * (x_ref[...] - hb)
        o_ref[...] = hn
        t_ref[...] = hn * dinv_ref[...]

    o = jax.ShapeDtypeStruct((NP, D), jnp.float32)
    return pl.pallas_call(
        body, grid=(NP // _RE,),
        in_specs=[_rb(_RE), _rb(_RE), _rb(_RE), _rb(_RE), _rb(_RE, 1),
                  _rb(_RE, 1), _rb(_RE), _rb(_RE), _rb(_RE, 1)],
        out_specs=[_rb(_RE)] * 2,
        out_shape=[o] * 2,
    )(u, w, h2, h2sq, cnt, cntc, h, x_, dinv)


def _tc_head(h, p):
    bn_sc = 1.0 / math.sqrt(1.0 + BN_EPS)

    def body(h_ref, w1, b1, w2, b2, w3, b3, w4, b4, g_ref, be_ref, o_ref):
        g = g_ref[...] * bn_sc
        be = be_ref[...]
        o = jnp.dot(h_ref[...], w1[...],
                    preferred_element_type=jnp.float32) + b1[...]
        o = jax.nn.relu(g * o + be)
        o = jnp.dot(o, w2[...], preferred_element_type=jnp.float32) + b2[...]
        o = jax.nn.relu(g * o + be)
        o = jnp.dot(o, w3[...], preferred_element_type=jnp.float32) + b3[...]
        o = jax.nn.relu(g * o + be)
        o_ref[...] = jnp.dot(o, w4[...],
                             preferred_element_type=jnp.float32) + b4[...]

    return pl.pallas_call(
        body, grid=(N // _RH,),
        in_specs=[_rb(_RH), _full((D, D)), _full((1, D)), _full((D, D)),
                  _full((1, D)), _full((D, D)), _full((1, D)),
                  _full((D, 1)), _full((1, 1)), _full((1, D)), _full((1, D))],
        out_specs=_rb(_RH, 1),
        out_shape=jax.ShapeDtypeStruct((N, 1), jnp.float32),
    )(h, p['lin1_w'], p['lin1_b'].reshape(1, D),
      p['lin2_w'], p['lin2_b'].reshape(1, D),
      p['lin3a_w'], p['lin3a_b'].reshape(1, D),
      p['lin4a_w'], p['lin4a_b'].reshape(1, 1),
      p['bn_g'].reshape(1, D), p['bn_b'].reshape(1, D))


def _tc_dirichlet(h, r0, r1, wsum):
    def body(h_ref, r0_ref, r1_ref, w_ref, o_ref):
        i = pl.program_id(0)
        hb = h_ref[...]
        part = (jnp.sum(w_ref[...] * jnp.sum(hb * hb, axis=1, keepdims=True))
                - 2.0 * jnp.sum(hb * (r0_ref[...] + r1_ref[...])))

        @pl.when(i == 0)
        def _():
            o_ref[...] = jnp.zeros((1, 1), jnp.float32)

        o_ref[...] += part

        @pl.when(i == pl.num_programs(0) - 1)
        def _():
            o_ref[...] = o_ref[...] * (0.5 / float(E))

    return pl.pallas_call(
        body, grid=(N // _RH,),
        in_specs=[_rb(_RH), _rb(_RH), _rb(_RH), _rb(_RH, 1)],
        out_specs=pl.BlockSpec((1, 1), lambda i: (0, 0)),
        out_shape=jax.ShapeDtypeStruct((1, 1), jnp.float32),
    )(h, r0, r1, wsum)


def kernel(x, edge_index, params):
    src = edge_index[0]
    dst = edge_index[1]
    pad = jnp.full((EP - E,), N, jnp.int32)
    srcp = jnp.concatenate([src, pad]).reshape(NSC, NCH, CH)
    dstp = jnp.concatenate([dst, pad]).reshape(NSC, NCH, CH)
    ifwd = jnp.stack([srcp, dstp], axis=2)
    irev = jnp.stack([dstp, srcp], axis=2)
    xp = jnp.pad(x, ((0, NP - N), (0, 0)))
    zblk = jnp.zeros((ROWS_T, D), jnp.float32)

    counts = _sc_degree(dstp, srcp, jnp.ones((CH, D), jnp.float32), zblk)
    dinv, cnt, cntc, wsum = _tc_degree(counts)

    h, t = _tc_pre(xp, params['pre_w'], params['pre_b'].reshape(1, D), dinv)
    for i in range(DEPTH):
        p0, p1 = _sc_split_propagate(t, ifwd, zblk)
        x_, h2, h2sq = _tc_layer1(p0, p1, t, dinv,
                                  params['conv_w'][i], params['gg_w'][i],
                                  params['conv_b'][i].reshape(1, D),
                                  params['gg_b'][i].reshape(1, D))
        u, w = _sc_dual_propagate(h2, h2sq, irev, zblk)
        h, t = _tc_epi2(u, w, h2, h2sq, cnt, cntc, h, x_, dinv)

    r0, r1 = _sc_split_propagate(h, irev, zblk)
    out = _tc_head(h, params)
    de = _tc_dirichlet(h, r0, r1, wsum)
    return (out, out, de[0, 0])
